# baseline (speedup 1.0000x reference)
import functools

import jax
import jax.numpy as jnp
from jax import lax
from jax.experimental import pallas as pl
from jax.experimental.pallas import tpu as pltpu
from jax.experimental.pallas import tpu_sc as plsc

L = 16
MAXLEN = 2048
CHUNK = 128
ZROWS = 64


def _tailzero_body(dur_ref, out_ref):
    jblk = pl.program_id(1)
    total = jnp.sum(dur_ref[...])
    limit = jnp.minimum(total, MAXLEN)

    @pl.when(jblk * CHUNK >= limit)
    def _():
        out_ref[...] = jnp.zeros_like(out_ref)


def _regulator_body(B, T, D, x_hbm, dur_hbm, out_hbm, mel_hbm,
                    dur_v, cum_v, start_v, arr_v, idx_v, mel_v,
                    rows0, rows1, rows2, gsem0, gsem1, gsem2):
    b = lax.axis_index("s")
    h = lax.axis_index("c")

    pltpu.sync_copy(dur_hbm.at[b], dur_v)

    iota = lax.iota(jnp.int32, L)

    def cum_step(i, carry):
        for u in range(2):
            v = dur_v[pl.ds((2 * i + u) * L, L)]
            s = plsc.cumsum(v) + carry
            cum_v[pl.ds((2 * i + u) * L, L)] = s
            start_v[pl.ds((2 * i + u) * L, L)] = s - v
            carry = s[L - 1]
        return carry

    total = lax.fori_loop(0, T // L // 2, cum_step, jnp.int32(0))

    zeros16 = jnp.zeros((L,), jnp.int32)

    limit = jnp.minimum(total, MAXLEN)
    kmax = jnp.minimum((limit // CHUNK + 1) * (CHUNK // L), MAXLEN // L)
    kmax2 = (kmax + 1) // 2

    def zero_step(i, carry):
        for u in range(2):
            arr_v[pl.ds((2 * i + u) * L, L)] = zeros16
        return carry

    lax.fori_loop(0, kmax2, zero_step, jnp.int32(0))

    def scatter_step(i, carry):
        for u in range(2):
            d = dur_v[pl.ds((2 * i + u) * L, L)]
            st = start_v[pl.ds((2 * i + u) * L, L)]
            ids = iota + (2 * i + u) * L
            plsc.store_scatter(arr_v, [st], ids, mask=(d > 0) & (st < MAXLEN))
        return carry

    lax.fori_loop(0, T // L // 2, scatter_step, jnp.int32(0))

    base = b * T

    def cummax_step(i, carry):
        for u in range(2):
            m = jnp.maximum(plsc.cummax(arr_v[pl.ds((2 * i + u) * L, L)]), carry)
            j = iota + (2 * i + u) * L
            idx_v[pl.ds((2 * i + u) * L, L)] = jnp.where(j < limit, m, 0) + base
            carry = m[L - 1]
        return carry

    lax.fori_loop(0, kmax2, cummax_step, jnp.int32(0))

    @pl.when(h == 0)
    def _():
        mel_v[...] = jnp.full((L,), total, jnp.int32)
        pltpu.sync_copy(mel_v, mel_hbm.at[b])

    zerosf = jnp.zeros((L,), jnp.float32)

    nrounds = MAXLEN // CHUNK // 2
    bufs = (rows0, rows1, rows2)
    sems = (gsem0, gsem1, gsem2)
    nb = len(bufs)
    h_eff = (h + b) % 2
    offs = [None] * nrounds
    conds = [None] * nrounds
    for k in range(nrounds):
        offs[k] = (2 * k + h_eff) * CHUNK
        conds[k] = offs[k] < limit

    copies = [None] * nrounds

    def start(k):
        @pl.when(conds[k])
        def _():
            pltpu.async_copy(
                x_hbm.at[idx_v.at[pl.ds(offs[k], CHUNK)]],
                bufs[k % nb], sems[k % nb])

    def wait(k):
        @pl.when(conds[k])
        def _():
            pltpu.make_async_copy(
                x_hbm.at[idx_v.at[pl.ds(offs[k], CHUNK)]],
                bufs[k % nb], sems[k % nb]).wait()

    start(0)
    start(1)
    for k in range(nrounds):
        wait(k)
        if k + 2 < nrounds:
            start(k + 2)

        @pl.when(conds[k])
        def _(k=k):
            r0 = jnp.clip(limit - offs[k], 0, CHUNK)

            def zrow_step(r, carry):
                for ci in range(D // L):
                    bufs[k % nb][r, pl.ds(ci * L, L)] = zerosf
                return carry

            lax.fori_loop(r0, CHUNK, zrow_step, jnp.int32(0))
            pltpu.sync_copy(bufs[k % nb], out_hbm.at[b, pl.ds(offs[k], CHUNK)])


def kernel(x, duration, max_len):
    B, T, D = x.shape
    xp = x.reshape(B * T, D)

    zeroed = pl.pallas_call(
        _tailzero_body,
        grid=(B, MAXLEN // CHUNK),
        in_specs=[pl.BlockSpec((1, 1, T), lambda i, j: (i, 0, 0))],
        out_specs=pl.BlockSpec((1, CHUNK, D), lambda i, j: (i, j, 0)),
        out_shape=jax.ShapeDtypeStruct((B, MAXLEN, D), x.dtype),
    )(duration.reshape(B, 1, T))

    out_ref = jax.new_ref(zeroed)
    body = functools.partial(_regulator_body, B, T, D)
    mel = pl.kernel(
        body,
        out_type=jax.ShapeDtypeStruct((B, L), jnp.int32),
        mesh=plsc.VectorSubcoreMesh(core_axis_name="c", subcore_axis_name="s"),
        compiler_params=pltpu.CompilerParams(needs_layout_passes=False),
        scratch_types=(
            pltpu.VMEM((T,), jnp.int32),
            pltpu.VMEM((T,), jnp.int32),
            pltpu.VMEM((T,), jnp.int32),
            pltpu.VMEM((MAXLEN,), jnp.int32),
            pltpu.VMEM((MAXLEN,), jnp.int32),
            pltpu.VMEM((L,), jnp.int32),
            pltpu.VMEM((CHUNK, D), jnp.float32),
            pltpu.VMEM((CHUNK, D), jnp.float32),
            pltpu.VMEM((CHUNK, D), jnp.float32),
            pltpu.SemaphoreType.DMA,
            pltpu.SemaphoreType.DMA,
            pltpu.SemaphoreType.DMA,
        ),
    )(xp, duration, out_ref)
    out = jax.freeze(out_ref)
    return out, mel[:, 0]


# speedup vs baseline: 2.4870x; 2.4870x over previous
import functools

import jax
import jax.numpy as jnp
from jax import lax
from jax.experimental import pallas as pl
from jax.experimental.pallas import tpu as pltpu
from jax.experimental.pallas import tpu_sc as plsc

L = 16
MAXLEN = 2048
CHUNK = 128
ZROWS = 64


def _regulator_body(B, T, D, x_hbm, dur_hbm, out_hbm, mel_hbm,
                    dur_v, cum_v, start_v, arr_v, idx_v, mel_v,
                    rows0, rows1, rows2, zbuf, gsem0, gsem1, gsem2):
    b = lax.axis_index("s")
    h = lax.axis_index("c")

    pltpu.sync_copy(dur_hbm.at[b], dur_v)

    iota = lax.iota(jnp.int32, L)

    def cum_step(i, carry):
        for u in range(2):
            v = dur_v[pl.ds((2 * i + u) * L, L)]
            s = plsc.cumsum(v) + carry
            cum_v[pl.ds((2 * i + u) * L, L)] = s
            start_v[pl.ds((2 * i + u) * L, L)] = s - v
            carry = s[L - 1]
        return carry

    total = lax.fori_loop(0, T // L // 2, cum_step, jnp.int32(0))

    zeros16 = jnp.zeros((L,), jnp.int32)

    limit = jnp.minimum(total, MAXLEN)
    kmax = jnp.minimum((limit // CHUNK + 1) * (CHUNK // L), MAXLEN // L)
    kmax2 = (kmax + 1) // 2

    def zero_step(i, carry):
        for u in range(2):
            arr_v[pl.ds((2 * i + u) * L, L)] = zeros16
        return carry

    lax.fori_loop(0, kmax2, zero_step, jnp.int32(0))

    def scatter_step(i, carry):
        for u in range(2):
            d = dur_v[pl.ds((2 * i + u) * L, L)]
            st = start_v[pl.ds((2 * i + u) * L, L)]
            ids = iota + (2 * i + u) * L
            plsc.store_scatter(arr_v, [st], ids, mask=(d > 0) & (st < MAXLEN))
        return carry

    lax.fori_loop(0, T // L // 2, scatter_step, jnp.int32(0))

    base = b * T

    def cummax_step(i, carry):
        for u in range(2):
            m = jnp.maximum(plsc.cummax(arr_v[pl.ds((2 * i + u) * L, L)]), carry)
            j = iota + (2 * i + u) * L
            idx_v[pl.ds((2 * i + u) * L, L)] = jnp.where(j < limit, m, 0) + base
            carry = m[L - 1]
        return carry

    lax.fori_loop(0, kmax2, cummax_step, jnp.int32(0))

    @pl.when(h == 0)
    def _():
        mel_v[...] = jnp.full((L,), total, jnp.int32)
        pltpu.sync_copy(mel_v, mel_hbm.at[b])

    zerosf = jnp.zeros((L,), jnp.float32)

    def zbuf_step(r, carry):
        for ci in range(D // L):
            zbuf[r, pl.ds(ci * L, L)] = zerosf
        return carry

    lax.fori_loop(0, ZROWS, zbuf_step, jnp.int32(0))

    nrounds = MAXLEN // CHUNK // 2
    bufs = (rows0, rows1, rows2)
    sems = (gsem0, gsem1, gsem2)
    nb = len(bufs)
    h_eff = (h + b) % 2
    offs = [None] * nrounds
    conds = [None] * nrounds
    for k in range(nrounds):
        offs[k] = (2 * k + h_eff) * CHUNK
        conds[k] = offs[k] < limit

    copies = [None] * nrounds

    def start(k):
        @pl.when(conds[k])
        def _():
            pltpu.async_copy(
                x_hbm.at[idx_v.at[pl.ds(offs[k], CHUNK)]],
                bufs[k % nb], sems[k % nb])

    def wait(k):
        @pl.when(conds[k])
        def _():
            pltpu.make_async_copy(
                x_hbm.at[idx_v.at[pl.ds(offs[k], CHUNK)]],
                bufs[k % nb], sems[k % nb]).wait()

    start(0)
    start(1)
    for k in range(nrounds):
        wait(k)
        if k + 2 < nrounds:
            start(k + 2)

        @pl.when(conds[k])
        def _(k=k):
            r0 = jnp.clip(limit - offs[k], 0, CHUNK)

            def zrow_step(r, carry):
                for ci in range(D // L):
                    bufs[k % nb][r, pl.ds(ci * L, L)] = zerosf
                return carry

            lax.fori_loop(r0, CHUNK, zrow_step, jnp.int32(0))
            pltpu.sync_copy(bufs[k % nb], out_hbm.at[b, pl.ds(offs[k], CHUNK)])

        @pl.when(jnp.logical_not(conds[k]))
        def _(k=k):
            for z in range(CHUNK // ZROWS):
                pltpu.sync_copy(
                    zbuf, out_hbm.at[b, pl.ds(offs[k] + z * ZROWS, ZROWS)])


def kernel(x, duration, max_len):
    B, T, D = x.shape
    xp = x.reshape(B * T, D)

    body = functools.partial(_regulator_body, B, T, D)
    out, mel = pl.kernel(
        body,
        out_type=(jax.ShapeDtypeStruct((B, MAXLEN, D), x.dtype),
                  jax.ShapeDtypeStruct((B, L), jnp.int32)),
        mesh=plsc.VectorSubcoreMesh(core_axis_name="c", subcore_axis_name="s"),
        compiler_params=pltpu.CompilerParams(needs_layout_passes=False),
        scratch_types=(
            pltpu.VMEM((T,), jnp.int32),
            pltpu.VMEM((T,), jnp.int32),
            pltpu.VMEM((T,), jnp.int32),
            pltpu.VMEM((MAXLEN,), jnp.int32),
            pltpu.VMEM((MAXLEN,), jnp.int32),
            pltpu.VMEM((L,), jnp.int32),
            pltpu.VMEM((CHUNK, D), jnp.float32),
            pltpu.VMEM((CHUNK, D), jnp.float32),
            pltpu.VMEM((CHUNK, D), jnp.float32),
            pltpu.VMEM((ZROWS, D), jnp.float32),
            pltpu.SemaphoreType.DMA,
            pltpu.SemaphoreType.DMA,
            pltpu.SemaphoreType.DMA,
        ),
    )(xp, duration)
    return out, mel[:, 0]
